# baseline (speedup 1.0000x reference)
import functools

import jax
import jax.numpy as jnp
from jax import lax
from jax.experimental import pallas as pl
from jax.experimental.pallas import tpu as pltpu
from jax.experimental.pallas import tpu_sc as plsc

NC, NS, L = 2, 16, 16
NW = NC * NS
N = 8192
H = 2048
HG = 1024
HQ = 512
TPW = N // NW
CH = L
NCHUNK = TPW // CH
UNROLL = 8
INV_H = 1.0 / H
EPS = 1e-12


def _rsqrt(x):
    xh = x * 0.5
    i = plsc.bitcast(x, jnp.int32)
    i = jnp.int32(0x5F3759DF) - (i >> 1)
    y = plsc.bitcast(i, jnp.float32)
    for _ in range(3):
        y = y * (1.5 - xh * y * y)
    return y


@functools.partial(
    pl.kernel,
    out_type=jax.ShapeDtypeStruct((N, H), jnp.float32),
    mesh=plsc.VectorSubcoreMesh(
        core_axis_name="c", subcore_axis_name="s", num_cores=NC, num_subcores=NS
    ),
    compiler_params=pltpu.CompilerParams(
        use_tc_tiling_on_sc=False, needs_layout_passes=False
    ),
    scratch_types=[
        pltpu.VMEM((TPW,), jnp.int32),
        pltpu.VMEM((TPW,), jnp.int32),
        pltpu.VMEM((TPW,), jnp.int32),
        pltpu.VMEM((CH, HG), jnp.float32),
        pltpu.VMEM((CH, HQ), jnp.float32),
        pltpu.VMEM((CH, HQ), jnp.float32),
        pltpu.VMEM((H,), jnp.float32),
        pltpu.VMEM((H,), jnp.float32),
        pltpu.SemaphoreType.DMA,
        pltpu.SemaphoreType.DMA,
        pltpu.SemaphoreType.DMA,
    ],
)
def _embed_ln(gate_hbm, qub_hbm, ids_hbm, tt_hbm, pos_hbm, gam_hbm, bet_hbm,
              out_hbm, ids_v, tt_v, pos_v, gbuf, q1buf, q2buf, gam_v, bet_v,
              sem0, sem1, sem2):
    wid = lax.axis_index("s") * NC + lax.axis_index("c")
    base = wid * TPW
    pltpu.sync_copy(gam_hbm, gam_v)
    pltpu.sync_copy(bet_hbm, bet_v)
    pltpu.sync_copy(ids_hbm.at[pl.ds(base, TPW)], ids_v)
    pltpu.sync_copy(tt_hbm.at[pl.ds(base, TPW)], tt_v)
    pltpu.sync_copy(pos_hbm.at[pl.ds(base, TPW)], pos_v)

    rows = lax.iota(jnp.int32, L)

    def reduce_body(buf):
        def body(jb, carry):
            a, a2 = carry
            for u in range(UNROLL):
                col = jnp.full((L,), jb * UNROLL + u, jnp.int32)
                x = plsc.load_gather(buf, [rows, col])
                a = a + x
                a2 = a2 + x * x
            return a, a2
        return body

    def norm_body(buf, coff, msp, isp):
        def body(jb, _):
            o = jb * L
            g = gam_v[pl.ds(coff + o, L)]
            b = bet_v[pl.ds(coff + o, L)]
            for t in range(L):
                x = buf[t, pl.ds(o, L)]
                y = (x - msp[t]) * isp[t] * g + b
                buf[t, pl.ds(o, L)] = y
            return 0
        return body

    def chunk_body(c, _):
        cs = c * CH
        cp0 = pltpu.async_copy(gate_hbm.at[ids_v.at[pl.ds(cs, CH)]], gbuf, sem0)
        cp1 = pltpu.async_copy(qub_hbm.at[tt_v.at[pl.ds(cs, CH)]], q1buf, sem1)
        cp2 = pltpu.async_copy(qub_hbm.at[pos_v.at[pl.ds(cs, CH)]], q2buf, sem2)
        cp0.wait()
        cp1.wait()
        cp2.wait()

        zero = jnp.zeros((L,), jnp.float32)
        acc, acc2 = lax.fori_loop(0, HG // UNROLL, reduce_body(gbuf), (zero, zero))
        acc, acc2 = lax.fori_loop(0, HQ // UNROLL, reduce_body(q1buf), (acc, acc2))
        acc, acc2 = lax.fori_loop(0, HQ // UNROLL, reduce_body(q2buf), (acc, acc2))
        mean = acc * INV_H
        var = acc2 * INV_H - mean * mean
        inv = _rsqrt(var + EPS)
        msp = [jnp.full((L,), mean[t]) for t in range(L)]
        isp = [jnp.full((L,), inv[t]) for t in range(L)]

        lax.fori_loop(0, HG // L, norm_body(gbuf, 0, msp, isp), 0)
        lax.fori_loop(0, HQ // L, norm_body(q1buf, HG, msp, isp), 0)
        lax.fori_loop(0, HQ // L, norm_body(q2buf, HG + HQ, msp, isp), 0)

        rb = base + cs
        pltpu.sync_copy(gbuf, out_hbm.at[pl.ds(rb, CH), pl.ds(0, HG)])
        pltpu.sync_copy(q1buf, out_hbm.at[pl.ds(rb, CH), pl.ds(HG, HQ)])
        pltpu.sync_copy(q2buf, out_hbm.at[pl.ds(rb, CH), pl.ds(HG + HQ, HQ)])
        return 0

    lax.fori_loop(0, NCHUNK, chunk_body, 0)


def kernel(input_ids, token_type_ids, position_ids, gate_table, qubit_table,
           ln_gamma, ln_beta):
    B, S = input_ids.shape
    ids = input_ids.reshape(-1).astype(jnp.int32)
    tts = token_type_ids.reshape(-1).astype(jnp.int32)
    pos = position_ids.reshape(-1).astype(jnp.int32)
    out = _embed_ln(gate_table, qubit_table, ids, tts, pos, ln_gamma, ln_beta)
    return out.reshape(B, S, H)
